# baseline (speedup 1.0000x reference)
import functools

import jax
import jax.numpy as jnp
from jax import lax
from jax.experimental import pallas as pl
from jax.experimental.pallas import tpu as pltpu
from jax.experimental.pallas import tpu_sc as plsc

NUM_GROUP = 4
BATCH = 256
N_OUT = BATCH // 2
LAM = 0.3
OML = 1.0 - LAM

CH = 512
CPI = (3 * 224 * 224) // CH
NROWS = BATCH * CPI

NC = 2
NS = 16
NW = NC * NS
IPW = N_OUT // NW
JPW = IPW * CPI
G = 56
NGROUPS = JPW // G
VPC = CH // 16

TW = 16
TPW = N_OUT // TW
TD = 1024


def _mix_body(img, tgt, aidx, bidx, taidx, tbidx, out_img, out_tgt,
              aidx_v, bidx_v, tidx_v, abuf, bbuf, tabuf, tbbuf, sem_a, sem_b):
    wid = lax.axis_index("s") * NC + lax.axis_index("c")

    @pl.when(wid < TW)
    def _targets():
        pltpu.sync_copy(taidx.at[wid], tidx_v.at[0])
        pltpu.sync_copy(tbidx.at[wid], tidx_v.at[1])
        ta = pltpu.async_copy(tgt.at[tidx_v.at[0]], tabuf, sem_a)
        tb = pltpu.async_copy(tgt.at[tidx_v.at[1]], tbbuf, sem_b)
        ta.wait()
        tb.wait()

        def _trow(r, carry):
            for v in range(TD // 16):
                sl = pl.ds(v * 16, 16)
                tabuf[r, sl] = tabuf[r, sl] * LAM + tbbuf[r, sl] * OML
            return carry

        lax.fori_loop(0, TPW, _trow, 0)
        pltpu.sync_copy(tabuf, out_tgt.at[pl.ds(wid * TPW, TPW)])

    pltpu.sync_copy(aidx.at[wid], aidx_v)
    pltpu.sync_copy(bidx.at[wid], bidx_v)

    def _group(g, carry):
        ca = pltpu.async_copy(img.at[aidx_v.at[g]], abuf, sem_a)
        cb = pltpu.async_copy(img.at[bidx_v.at[g]], bbuf, sem_b)
        ca.wait()
        cb.wait()

        def _row(r, c2):
            for v in range(VPC):
                sl = pl.ds(v * 16, 16)
                abuf[r, sl] = abuf[r, sl] * LAM + bbuf[r, sl] * OML
            return c2

        lax.fori_loop(0, G, _row, 0)
        pltpu.sync_copy(abuf, out_img.at[pl.ds(wid * JPW + g * G, G)])
        return carry

    lax.fori_loop(0, NGROUPS, _group, 0)


@jax.jit
def _mixup(batch_image, batch_target, batch_group):
    order = jnp.argsort(batch_group, stable=True)
    idx0 = order[:N_OUT]
    idx1 = order[N_OUT:]
    perm0 = jax.random.permutation(jax.random.key(1), N_OUT)
    perm1 = jax.random.permutation(jax.random.key(2), N_OUT)
    g0 = idx0[perm0].astype(jnp.int32)
    g1 = idx1[perm1].astype(jnp.int32)

    chunk = jnp.arange(CPI, dtype=jnp.int32)
    a_rows = (g0[:, None] * CPI + chunk[None, :]).reshape(NW, NGROUPS, G)
    b_rows = (g1[:, None] * CPI + chunk[None, :]).reshape(NW, NGROUPS, G)
    ta_rows = g0.reshape(TW, TPW)
    tb_rows = g1.reshape(TW, TPW)

    img2d = batch_image.reshape(NROWS, CH)
    tgt_pad = jnp.pad(batch_target, ((0, 0), (0, TD - batch_target.shape[1])))

    mesh = plsc.VectorSubcoreMesh(core_axis_name="c", subcore_axis_name="s")
    run = functools.partial(
        pl.kernel,
        mesh=mesh,
        compiler_params=pltpu.CompilerParams(use_tc_tiling_on_sc=False),
        out_type=(
            jax.ShapeDtypeStruct((N_OUT * CPI, CH), jnp.float32),
            jax.ShapeDtypeStruct((N_OUT, TD), jnp.float32),
        ),
        scratch_types=[
            pltpu.VMEM((NGROUPS, G), jnp.int32),
            pltpu.VMEM((NGROUPS, G), jnp.int32),
            pltpu.VMEM((2, TPW), jnp.int32),
            pltpu.VMEM((G, CH), jnp.float32),
            pltpu.VMEM((G, CH), jnp.float32),
            pltpu.VMEM((TPW, TD), jnp.float32),
            pltpu.VMEM((TPW, TD), jnp.float32),
            pltpu.SemaphoreType.DMA,
            pltpu.SemaphoreType.DMA,
        ],
    )(_mix_body)
    out_img, out_tgt = run(img2d, tgt_pad, a_rows, b_rows, ta_rows, tb_rows)

    inputs_mix = out_img.reshape(N_OUT, 3, 224, 224)
    targets_mix = out_tgt[:, : batch_target.shape[1]]
    return inputs_mix, targets_mix


def kernel(batch_image, batch_target, batch_group):
    return _mixup(batch_image, batch_target, batch_group)


# speedup vs baseline: 1.5621x; 1.5621x over previous
import functools

import jax
import jax.numpy as jnp
from jax import lax
from jax.experimental import pallas as pl
from jax.experimental.pallas import tpu as pltpu
from jax.experimental.pallas import tpu_sc as plsc

NUM_GROUP = 4
BATCH = 256
N_OUT = BATCH // 2
LAM = 0.3
OML = 1.0 - LAM

SH = 8
SW = 224
SPI = 3 * 224 // SH
NSLAB = BATCH * SPI
OSLAB = N_OUT * SPI

NC = 2
NS = 16
NW = NC * NS
IPW = N_OUT // NW
RUN = 14
NRUNS = SPI // RUN
VPR = SW // 16

TW = 16
TPW = N_OUT // TW
TD = 1024


def _img_body(img, gsrc, out_img, gsrc_v, abuf, bbuf, sem_a, sem_b):
    wid = lax.axis_index("s") * NC + lax.axis_index("c")

    pltpu.sync_copy(gsrc.at[wid], gsrc_v)
    gv = gsrc_v[0, pl.ds(0, 16)]
    lanes = jax.lax.iota(jnp.int32, 16)

    for k in range(IPW):
        ga = jnp.sum(jnp.where(lanes == k, gv, 0))
        gb = jnp.sum(jnp.where(lanes == (IPW + k), gv, 0))

        def _run(r, carry):
            base_a = ga * SPI + r * RUN
            base_b = gb * SPI + r * RUN
            ca = pltpu.async_copy(img.at[pl.ds(base_a, RUN)], abuf, sem_a)
            cb = pltpu.async_copy(img.at[pl.ds(base_b, RUN)], bbuf, sem_b)
            ca.wait()
            cb.wait()

            def _slab(s, c2):
                for rr in range(SH):
                    for v in range(VPR):
                        sl = pl.ds(v * 16, 16)
                        abuf[s, rr, sl] = (abuf[s, rr, sl] * LAM
                                           + bbuf[s, rr, sl] * OML)
                return c2

            lax.fori_loop(0, RUN, _slab, 0)
            out_base = (wid * IPW + k) * SPI + r * RUN
            pltpu.sync_copy(abuf, out_img.at[pl.ds(out_base, RUN)])
            return carry

        lax.fori_loop(0, NRUNS, _run, 0)


def _tgt_body(tgt, taidx, tbidx, out_tgt, tidx_v, tabuf, tbbuf, sem_a, sem_b):
    wid = lax.axis_index("s") * NC + lax.axis_index("c")

    @pl.when(wid < TW)
    def _targets():
        pltpu.sync_copy(taidx.at[wid], tidx_v.at[0])
        pltpu.sync_copy(tbidx.at[wid], tidx_v.at[1])
        ta = pltpu.async_copy(tgt.at[tidx_v.at[0]], tabuf, sem_a)
        tb = pltpu.async_copy(tgt.at[tidx_v.at[1]], tbbuf, sem_b)
        ta.wait()
        tb.wait()

        def _trow(r, carry):
            for v in range(TD // 16):
                sl = pl.ds(v * 16, 16)
                tabuf[r, sl] = tabuf[r, sl] * LAM + tbbuf[r, sl] * OML
            return carry

        lax.fori_loop(0, TPW, _trow, 0)
        pltpu.sync_copy(tabuf, out_tgt.at[pl.ds(wid * TPW, TPW)])


@jax.jit
def _mixup(batch_image, batch_target, batch_group):
    order = jnp.argsort(batch_group, stable=True)
    idx0 = order[:N_OUT]
    idx1 = order[N_OUT:]
    perm0 = jax.random.permutation(jax.random.key(1), N_OUT)
    perm1 = jax.random.permutation(jax.random.key(2), N_OUT)
    g0 = idx0[perm0].astype(jnp.int32)
    g1 = idx1[perm1].astype(jnp.int32)

    gsrc = jnp.zeros((NW, 8, 128), jnp.int32)
    gsrc = gsrc.at[:, 0, :IPW].set(g0.reshape(NW, IPW))
    gsrc = gsrc.at[:, 0, IPW:2 * IPW].set(g1.reshape(NW, IPW))
    ta_rows = g0.reshape(TW, TPW)
    tb_rows = g1.reshape(TW, TPW)

    img3 = batch_image.reshape(NSLAB, SH, SW)
    tgt_pad = jnp.pad(batch_target, ((0, 0), (0, TD - batch_target.shape[1])))

    mesh = plsc.VectorSubcoreMesh(core_axis_name="c", subcore_axis_name="s")

    out_img = functools.partial(
        pl.kernel,
        mesh=mesh,
        compiler_params=pltpu.CompilerParams(use_tc_tiling_on_sc=True,
                                             needs_layout_passes=False),
        out_type=jax.ShapeDtypeStruct((OSLAB, SH, SW), jnp.float32),
        scratch_types=[
            pltpu.VMEM((8, 128), jnp.int32),
            pltpu.VMEM((RUN, SH, SW), jnp.float32),
            pltpu.VMEM((RUN, SH, SW), jnp.float32),
            pltpu.SemaphoreType.DMA,
            pltpu.SemaphoreType.DMA,
        ],
    )(_img_body)(img3, gsrc)

    out_tgt = functools.partial(
        pl.kernel,
        mesh=mesh,
        compiler_params=pltpu.CompilerParams(use_tc_tiling_on_sc=False),
        out_type=jax.ShapeDtypeStruct((N_OUT, TD), jnp.float32),
        scratch_types=[
            pltpu.VMEM((2, TPW), jnp.int32),
            pltpu.VMEM((TPW, TD), jnp.float32),
            pltpu.VMEM((TPW, TD), jnp.float32),
            pltpu.SemaphoreType.DMA,
            pltpu.SemaphoreType.DMA,
        ],
    )(_tgt_body)(tgt_pad, ta_rows, tb_rows)

    inputs_mix = out_img.reshape(N_OUT, 3, 224, 224)
    targets_mix = out_tgt[:, : batch_target.shape[1]]
    return inputs_mix, targets_mix


def kernel(batch_image, batch_target, batch_group):
    return _mixup(batch_image, batch_target, batch_group)
